# baseline (speedup 1.0000x reference)
import functools

import jax
import jax.numpy as jnp
from jax import lax
from jax.experimental import pallas as pl
from jax.experimental.pallas import tpu as pltpu
from jax.experimental.pallas import tpu_sc as plsc

NUM_ROWS = 1_000_000
D = 32
B = 16_384
NC, NS = 2, 16
NW = NC * NS
CPW = B // NW
IDX_COLS = 128
IDX_ROWS = B // IDX_COLS
ROWS_PW = IDX_ROWS // NW


def _worker_id():
  return lax.axis_index("s") * NC + lax.axis_index("c")


@functools.cache
def _sc_kernels():
  mesh = plsc.VectorSubcoreMesh(
      core_axis_name="c", subcore_axis_name="s", num_cores=NC, num_subcores=NS)
  params = pltpu.CompilerParams(use_tc_tiling_on_sc=False)

  @functools.partial(
      pl.kernel,
      out_type=jax.ShapeDtypeStruct((D, B), jnp.float32),
      mesh=mesh,
      scratch_types=[
          pltpu.VMEM((ROWS_PW, IDX_COLS), jnp.int32),
          pltpu.VMEM((D, CPW), jnp.float32),
          pltpu.VMEM((D, CPW), jnp.float32),
          pltpu.SemaphoreType.DMA,
      ],
      compiler_params=params,
  )
  def sc_gather_blend(mem_hbm, idx_hbm, lg_hbm, tgt_hbm, idx_v, t_v, lg_v,
                      sem):
    w = _worker_id()
    c0 = w * CPW
    pltpu.sync_copy(idx_hbm.at[pl.ds(w * ROWS_PW, ROWS_PW)], idx_v)
    pltpu.sync_copy(lg_hbm.at[:, pl.ds(c0, CPW)], lg_v)
    handles = [
        pltpu.async_copy(
            mem_hbm.at[c].at[idx_v.at[j]],
            t_v.at[c, pl.ds(j * IDX_COLS, IDX_COLS)],
            sem,
        )
        for c in range(D)
        for j in range(ROWS_PW)
    ]
    for h in handles:
      h.wait()
    for c in range(D):
      @pl.loop(0, CPW // 16)
      def _(k, c=c):
        sl = pl.ds(k * 16, 16)
        x = lg_v[c, sl]
        t_v[c, sl] = 0.9 * t_v[c, sl] + 0.1 / (1.0 + jnp.exp(-x))
    pltpu.sync_copy(t_v, tgt_hbm.at[:, pl.ds(c0, CPW)])

  @functools.partial(
      pl.kernel,
      out_type=(),
      mesh=mesh,
      scratch_types=[
          pltpu.VMEM((ROWS_PW, IDX_COLS), jnp.int32),
          pltpu.VMEM((D, CPW), jnp.float32),
          pltpu.SemaphoreType.DMA,
      ],
      compiler_params=params,
  )
  def sc_scatter(mem_hbm, idx_hbm, tgt_hbm, idx_v, t_v, sem):
    w = _worker_id()
    c0 = w * CPW
    pltpu.sync_copy(idx_hbm.at[pl.ds(w * ROWS_PW, ROWS_PW)], idx_v)
    pltpu.sync_copy(tgt_hbm.at[:, pl.ds(c0, CPW)], t_v)
    handles = [
        pltpu.async_copy(
            t_v.at[c, pl.ds(j * IDX_COLS, IDX_COLS)],
            mem_hbm.at[c].at[idx_v.at[j]],
            sem,
        )
        for c in range(D)
        for j in range(ROWS_PW)
    ]
    for h in handles:
      h.wait()

  return sc_gather_blend, sc_scatter


_BLK = 2048
_NBLK = B // _BLK


def _loss_body(l_ref, tg_ref, loss_ref):
  i = pl.program_id(0)
  l = l_ref[...]
  part = jnp.sum(
      jnp.maximum(l, 0.0) - l * tg_ref[...] + jnp.log1p(jnp.exp(-jnp.abs(l))))

  @pl.when(i == 0)
  def _():
    loss_ref[0, 0] = 0.0

  loss_ref[0, 0] += part

  @pl.when(i == _NBLK - 1)
  def _():
    loss_ref[0, 0] = loss_ref[0, 0] / (B * D)


_loss = pl.pallas_call(
    _loss_body,
    grid=(_NBLK,),
    in_specs=[
        pl.BlockSpec((D, _BLK), lambda i: (0, i)),
        pl.BlockSpec((D, _BLK), lambda i: (0, i)),
    ],
    out_specs=pl.BlockSpec(memory_space=pltpu.SMEM),
    out_shape=jax.ShapeDtypeStruct((1, 1), jnp.float32),
)


def kernel(logits, labels, index, targets_mem):
  del labels
  sc_gather_blend, sc_scatter = _sc_kernels()
  bank_t = targets_mem.T
  logits_t = logits.T
  idx2d = index.reshape(IDX_ROWS, IDX_COLS)
  mem_ref = jax.new_ref(bank_t)
  targets_t = sc_gather_blend(mem_ref, idx2d, logits_t)
  loss = _loss(logits_t, targets_t)
  sc_scatter(mem_ref, idx2d, targets_t)
  new_mem = mem_ref[...].T
  return loss[0, 0], new_mem


# speedup vs baseline: 14.1848x; 14.1848x over previous
import functools

import jax
import jax.numpy as jnp
from jax import lax
from jax.experimental import pallas as pl
from jax.experimental.pallas import tpu as pltpu
from jax.experimental.pallas import tpu_sc as plsc

NUM_ROWS = 1_000_000
D = 32
B = 16_384
NC, NS = 2, 16
NW = NC * NS
W = 512
N_FULL = NUM_ROWS // W
TAIL = NUM_ROWS - N_FULL * W
TAIL_Q = N_FULL
TAIL_W = TAIL_Q % NW
_SHIFT = W.bit_length() - 1
L16 = 16
CAP_W = 4096
CAP_C = 2048


def _iota16():
  return lax.iota(jnp.int32, 16)


@functools.cache
def _sc_merge():
  mesh = plsc.VectorSubcoreMesh(
      core_axis_name="c", subcore_axis_name="s", num_cores=NC, num_subcores=NS)

  @functools.partial(
      pl.kernel,
      out_type=(
          jax.ShapeDtypeStruct((D, NUM_ROWS), jnp.float32),
          jax.ShapeDtypeStruct((NC, NS, L16), jnp.float32),
      ),
      mesh=mesh,
      compiler_params=pltpu.CompilerParams(needs_layout_passes=False),
      scratch_types=[
          pltpu.VMEM((B // 128, 128), jnp.int32),
          pltpu.VMEM((CAP_W + 16,), jnp.int32),
          pltpu.VMEM((CAP_W + 16,), jnp.int32),
          pltpu.VMEM((CAP_C + 128,), jnp.int32),
          pltpu.VMEM((CAP_C + 128,), jnp.int32),
          pltpu.VMEM((D, W + 8), jnp.float32),
          pltpu.VMEM((16, 128), jnp.float32),
          pltpu.VMEM((16, 128), jnp.float32),
          pltpu.VMEM((L16,), jnp.float32),
          pltpu.SemaphoreType.DMA,
          pltpu.VMEM_SHARED((NS, L16), jnp.float32),
      ],
  )
  def merge(bank, idx, p01_lines, l_lines, out, dots, idx_all, r_hits, b_hits,
            ch_r, ch_b, buf, stg_p, stg_l, acc_v, stg_sem, shd):
    c = lax.axis_index("c")
    s = lax.axis_index("s")
    w = s * NC + c

    pltpu.sync_copy(idx, idx_all)

    zeros16i = jnp.zeros((16,), jnp.int32)
    sent16 = jnp.full((16,), jnp.int32(0x7FFFFFFF), jnp.int32)

    @pl.loop(0, (CAP_W + 16) // 16)
    def _(i):
      r_hits[pl.ds(i * 16, 16)] = sent16

    @pl.loop(0, (CAP_C + 128) // 16)
    def _(i):
      ch_r[pl.ds(i * 16, 16)] = zeros16i
      ch_b[pl.ds(i * 16, 16)] = zeros16i

    acc_v[...] = jnp.zeros((L16,), jnp.float32)

    def _compact(dst_a, dst_b, val_a, val_b, m, cnt, cap):
      mi = m.astype(jnp.int32)
      pos = cnt + jnp.cumsum(mi) - 1
      keep = m & (pos < cap)
      pos = jnp.where(keep, pos, jnp.full((16,), cap, jnp.int32))
      plsc.store_scatter(dst_a, [pos], val_a)
      plsc.store_scatter(dst_b, [pos], val_b)
      return jnp.minimum(cnt + jnp.sum(mi), cap)

    @pl.loop(0, B // 16, init_carry=jnp.int32(0))
    def n_w(i, cnt):
      row = lax.shift_right_logical(i, 3)
      col = (i % 8) * 16
      v = idx_all[row, pl.ds(col, 16)]
      m = (lax.shift_right_logical(v, _SHIFT) % NW) == w
      pos = _iota16() + i * 16
      return _compact(r_hits, b_hits, v, pos, m, cnt, CAP_W)

    plsc.subcore_barrier()

    ngrp = (n_w + 15) // 16

    def process_chunk(q, acc0):
      @pl.loop(0, ngrp, init_carry=jnp.int32(0))
      def m(i, cnt):
        r16 = r_hits[pl.ds(i * 16, 16)]
        b16 = b_hits[pl.ds(i * 16, 16)]
        mm = lax.shift_right_logical(r16, _SHIFT) == q
        return _compact(ch_r, ch_b, r16 % W, b16, mm, cnt, CAP_C)

      @pl.loop(0, (m + 15) // 16, init_carry=acc0)
      def acc(p, a):
        base = p * 16
        rem = m - base
        r16 = ch_r[pl.ds(base, 16)]
        b16 = ch_b[pl.ds(base, 16)]
        rows = _iota16()
        msk = rows < rem
        gvec = lax.shift_right_logical(b16, 2)
        h1 = pltpu.async_copy(p01_lines.at[gvec], stg_p, stg_sem)
        h2 = pltpu.async_copy(l_lines.at[gvec], stg_l, stg_sem)
        h1.wait()
        h2.wait()
        r16s = jnp.where(msk, r16, jnp.full((16,), W, jnp.int32))
        cbase = (b16 % 4) * D
        ag = a
        for cc in range(D):
          cvec = jnp.full((16,), cc, jnp.int32)
          old = plsc.load_gather(buf, [cvec, r16s])
          pv = plsc.load_gather(stg_p, [rows, cbase + cc])
          lv = plsc.load_gather(stg_l, [rows, cbase + cc])
          t = jnp.where(msk, 0.9 * old + pv, old)
          plsc.store_scatter(buf, [cvec, r16s], t)
          ag = ag + jnp.where(msk, lv * t, 0.0)
        return ag

      return acc

    nq = jnp.where(w < (N_FULL % NW), N_FULL // NW + 1, N_FULL // NW)

    @pl.loop(0, nq)
    def _(j):
      q = w + NW * j
      col0 = pl.multiple_of(q * W, W)
      pltpu.sync_copy(bank.at[:, pl.ds(col0, W)], buf.at[:, pl.ds(0, W)])
      acc_v[...] = process_chunk(q, acc_v[...])
      pltpu.sync_copy(buf.at[:, pl.ds(0, W)], out.at[:, pl.ds(col0, W)])

    pltpu.sync_copy(acc_v, shd.at[s])
    plsc.subcore_barrier()

    @pl.when(s == 0)
    def _():
      pltpu.sync_copy(shd, dots.at[c])

  return merge


def _prep_body(lt_ref, p_ref, l_ref):
  lt = lt_ref[...]
  p_ref[...] = (0.1 * jax.nn.sigmoid(lt)).T
  l_ref[...] = lt.T


_prep = pl.pallas_call(
    _prep_body,
    in_specs=[pl.BlockSpec((D, B), lambda: (0, 0))],
    out_specs=[
        pl.BlockSpec((B, D), lambda: (0, 0)),
        pl.BlockSpec((B, D), lambda: (0, 0)),
    ],
    out_shape=[
        jax.ShapeDtypeStruct((B, D), jnp.float32),
        jax.ShapeDtypeStruct((B, D), jnp.float32),
    ],
)


_TAIL_BLK = TAIL_Q * W // 128


def _tail_body(mem_ref, src_ref, idx_ref, p_ref, l_ref, out_ref, dot_ref):
  del mem_ref
  old = src_ref[...]
  colmask = lax.broadcasted_iota(jnp.int32, (1, 128), 1) < TAIL
  old = jnp.where(jnp.broadcast_to(colmask, (D, 128)), old, 0.0)
  cols = TAIL_Q * W + lax.broadcasted_iota(jnp.int32, (1, 128), 1)
  idx = idx_ref[...]
  m_hit = (idx == cols).astype(jnp.float32)
  p01 = p_ref[...]
  lg = l_ref[...]
  contract = (((0,), (0,)), ((), ()))
  p_sel = lax.dot_general(m_hit, p01, contract)
  lm = lax.dot_general(m_hit, lg, contract)
  anyhit = jnp.max(m_hit, axis=0, keepdims=True)
  blended = 0.9 * old + p_sel.T
  out_ref[...] = jnp.where(
      jnp.broadcast_to(anyhit > 0.0, (D, 128)), blended, old)
  term1 = jnp.sum(old * lm.T)
  anyhit_b = (idx >= TAIL_Q * W).astype(jnp.float32)
  s01 = jnp.sum(lg * p01, axis=1, keepdims=True)
  dot_ref[0, 0] = 0.9 * term1 + jnp.sum(anyhit_b * s01)


_tail_fix = pl.pallas_call(
    _tail_body,
    grid=(1,),
    in_specs=[
        pl.BlockSpec((D, 128), lambda i: (0, _TAIL_BLK)),
        pl.BlockSpec((D, 128), lambda i: (0, _TAIL_BLK)),
        pl.BlockSpec((B, 1), lambda i: (0, 0)),
        pl.BlockSpec((B, D), lambda i: (0, 0)),
        pl.BlockSpec((B, D), lambda i: (0, 0)),
    ],
    out_specs=[
        pl.BlockSpec((D, 128), lambda i: (0, _TAIL_BLK)),
        pl.BlockSpec(memory_space=pltpu.SMEM),
    ],
    out_shape=[
        jax.ShapeDtypeStruct((D, NUM_ROWS), jnp.float32),
        jax.ShapeDtypeStruct((1, 1), jnp.float32),
    ],
    input_output_aliases={0: 0},
)


_BLK = 2048
_NBLK = B // _BLK


def _loss_body(l_ref, dots_ref, dt_ref, loss_ref):
  i = pl.program_id(0)
  l = l_ref[...]
  part = jnp.sum(jnp.maximum(l, 0.0) + jnp.log1p(jnp.exp(-jnp.abs(l))))

  @pl.when(i == 0)
  def _():
    loss_ref[0, 0] = 0.0

  loss_ref[0, 0] += part

  @pl.when(i == _NBLK - 1)
  def _():
    loss_ref[0, 0] = (
        loss_ref[0, 0] - jnp.sum(dots_ref[...]) - dt_ref[0, 0]) / (B * D)


_loss = pl.pallas_call(
    _loss_body,
    grid=(_NBLK,),
    in_specs=[
        pl.BlockSpec((D, _BLK), lambda i: (0, i)),
        pl.BlockSpec((NC, NS, L16), lambda i: (0, 0, 0)),
        pl.BlockSpec(memory_space=pltpu.SMEM),
    ],
    out_specs=pl.BlockSpec(memory_space=pltpu.SMEM),
    out_shape=jax.ShapeDtypeStruct((1, 1), jnp.float32),
)


def kernel(logits, labels, index, targets_mem):
  del labels
  merge = _sc_merge()
  bank_t = targets_mem.T
  logits_t = logits.T
  p01_rows, l_rows = _prep(logits_t)
  new_bank, dots = merge(bank_t, index.reshape(B // 128, 128),
                         p01_rows.reshape(B * D // 128, 128),
                         l_rows.reshape(B * D // 128, 128))
  new_bank, dot_tail = _tail_fix(
      new_bank, bank_t, index.reshape(B, 1), p01_rows, l_rows)
  loss = _loss(logits_t, dots, dot_tail)
  return loss[0, 0], new_bank.T


# speedup vs baseline: 14.8793x; 1.0490x over previous
import functools

import jax
import jax.numpy as jnp
from jax import lax
from jax.experimental import pallas as pl
from jax.experimental.pallas import tpu as pltpu
from jax.experimental.pallas import tpu_sc as plsc

NUM_ROWS = 1_000_000
D = 32
B = 16_384
NC, NS = 2, 16
NW = NC * NS
W = 512
N_FULL = NUM_ROWS // W
TAIL = NUM_ROWS - N_FULL * W
TAIL_Q = N_FULL
TAIL_W = TAIL_Q % NW
_SHIFT = W.bit_length() - 1
L16 = 16
CAP_W = 4096
CAP_C = 2048


def _iota16():
  return lax.iota(jnp.int32, 16)


@functools.cache
def _sc_merge():
  mesh = plsc.VectorSubcoreMesh(
      core_axis_name="c", subcore_axis_name="s", num_cores=NC, num_subcores=NS)

  @functools.partial(
      pl.kernel,
      out_type=(
          jax.ShapeDtypeStruct((D, NUM_ROWS), jnp.float32),
          jax.ShapeDtypeStruct((NC, NS, L16), jnp.float32),
      ),
      mesh=mesh,
      compiler_params=pltpu.CompilerParams(needs_layout_passes=False),
      scratch_types=[
          pltpu.VMEM((B // 128, 128), jnp.int32),
          pltpu.VMEM((CAP_W + 16,), jnp.int32),
          pltpu.VMEM((CAP_W + 16,), jnp.int32),
          pltpu.VMEM((CAP_C + 128,), jnp.int32),
          pltpu.VMEM((CAP_C + 128,), jnp.int32),
          pltpu.VMEM((D, W + 8), jnp.float32),
          pltpu.VMEM((D, W + 8), jnp.float32),
          pltpu.VMEM((16, 128), jnp.float32),
          pltpu.VMEM((16, 128), jnp.float32),
          pltpu.VMEM((L16,), jnp.float32),
          pltpu.SemaphoreType.DMA,
          pltpu.SemaphoreType.DMA,
          pltpu.SemaphoreType.DMA,
          pltpu.SemaphoreType.DMA,
          pltpu.SemaphoreType.DMA,
          pltpu.VMEM_SHARED((NS, L16), jnp.float32),
      ],
  )
  def merge(bank, idx, p01_lines, l_lines, out, dots, idx_all, r_hits, b_hits,
            ch_r, ch_b, buf_a, buf_b, stg_p, stg_l, acc_v, stg_sem,
            in_sem_a, in_sem_b, out_sem_a, out_sem_b, shd):
    c = lax.axis_index("c")
    s = lax.axis_index("s")
    w = s * NC + c

    pltpu.sync_copy(idx, idx_all)

    zeros16i = jnp.zeros((16,), jnp.int32)
    sent16 = jnp.full((16,), jnp.int32(0x7FFFFFFF), jnp.int32)

    @pl.loop(0, (CAP_W + 16) // 16)
    def _(i):
      r_hits[pl.ds(i * 16, 16)] = sent16

    @pl.loop(0, (CAP_C + 128) // 16)
    def _(i):
      ch_r[pl.ds(i * 16, 16)] = zeros16i
      ch_b[pl.ds(i * 16, 16)] = zeros16i

    acc_v[...] = jnp.zeros((L16,), jnp.float32)

    def _compact(dst_a, dst_b, val_a, val_b, m, cnt, cap):
      mi = m.astype(jnp.int32)
      pos = cnt + jnp.cumsum(mi) - 1
      keep = m & (pos < cap)
      pos = jnp.where(keep, pos, jnp.full((16,), cap, jnp.int32))
      plsc.store_scatter(dst_a, [pos], val_a)
      plsc.store_scatter(dst_b, [pos], val_b)
      return jnp.minimum(cnt + jnp.sum(mi), cap)

    @pl.loop(0, B // 16, init_carry=jnp.int32(0))
    def n_w(i, cnt):
      row = lax.shift_right_logical(i, 3)
      col = (i % 8) * 16
      v = idx_all[row, pl.ds(col, 16)]
      m = (lax.shift_right_logical(v, _SHIFT) % NW) == w
      pos = _iota16() + i * 16
      return _compact(r_hits, b_hits, v, pos, m, cnt, CAP_W)

    plsc.subcore_barrier()

    ngrp = (n_w + 15) // 16

    def process_chunk(q, acc0, buf):
      @pl.loop(0, ngrp, init_carry=jnp.int32(0))
      def m(i, cnt):
        r16 = r_hits[pl.ds(i * 16, 16)]
        b16 = b_hits[pl.ds(i * 16, 16)]
        mm = lax.shift_right_logical(r16, _SHIFT) == q
        return _compact(ch_r, ch_b, r16 % W, b16, mm, cnt, CAP_C)

      @pl.loop(0, (m + 15) // 16, init_carry=acc0)
      def acc(p, a):
        base = p * 16
        rem = m - base
        r16 = ch_r[pl.ds(base, 16)]
        b16 = ch_b[pl.ds(base, 16)]
        rows = _iota16()
        msk = rows < rem
        gvec = lax.shift_right_logical(b16, 2)
        h1 = pltpu.async_copy(p01_lines.at[gvec], stg_p, stg_sem)
        h2 = pltpu.async_copy(l_lines.at[gvec], stg_l, stg_sem)
        h1.wait()
        h2.wait()
        r16s = jnp.where(msk, r16, jnp.full((16,), W, jnp.int32))
        cbase = (b16 % 4) * D
        ag = a
        for cc in range(D):
          cvec = jnp.full((16,), cc, jnp.int32)
          old = plsc.load_gather(buf, [cvec, r16s])
          pv = plsc.load_gather(stg_p, [rows, cbase + cc])
          lv = plsc.load_gather(stg_l, [rows, cbase + cc])
          t = jnp.where(msk, 0.9 * old + pv, old)
          plsc.store_scatter(buf, [cvec, r16s], t)
          ag = ag + jnp.where(msk, lv * t, 0.0)
        return ag

      return acc

    nq = jnp.where(w < (N_FULL % NW), N_FULL // NW + 1, N_FULL // NW)

    def _start_in(j, buf, sem):
      col0 = pl.multiple_of((w + NW * j) * W, W)
      pltpu.async_copy(bank.at[:, pl.ds(col0, W)], buf.at[:, pl.ds(0, W)],
                       sem)

    def _start_out(j, buf, sem):
      col0 = pl.multiple_of((w + NW * j) * W, W)
      pltpu.async_copy(buf.at[:, pl.ds(0, W)], out.at[:, pl.ds(col0, W)],
                       sem)

    def _drain(buf, sem):
      pltpu.make_async_copy(
          bank.at[:, pl.ds(0, W)], buf.at[:, pl.ds(0, W)], sem).wait()

    _start_in(jnp.int32(0), buf_a, in_sem_a)

    @pl.loop(0, nq)
    def _(j):
      even = (j % 2) == 0

      def _step(buf_c, in_c, out_c, buf_o, in_o, out_o):
        @pl.when(j + 1 < nq)
        def _():
          @pl.when(j >= 1)
          def _():
            _drain(buf_o, out_o)

          _start_in(j + 1, buf_o, in_o)

        _drain(buf_c, in_c)
        acc_v[...] = process_chunk(w + NW * j, acc_v[...], buf_c)
        _start_out(j, buf_c, out_c)

      @pl.when(even)
      def _():
        _step(buf_a, in_sem_a, out_sem_a, buf_b, in_sem_b, out_sem_b)

      @pl.when(jnp.logical_not(even))
      def _():
        _step(buf_b, in_sem_b, out_sem_b, buf_a, in_sem_a, out_sem_a)

    _drain(buf_a, out_sem_a)
    _drain(buf_b, out_sem_b)

    pltpu.sync_copy(acc_v, shd.at[s])
    plsc.subcore_barrier()

    @pl.when(s == 0)
    def _():
      pltpu.sync_copy(shd, dots.at[c])

  return merge


def _prep_body(lt_ref, p_ref, l_ref):
  lt = lt_ref[...]
  p_ref[...] = (0.1 * jax.nn.sigmoid(lt)).T
  l_ref[...] = lt.T


_prep = pl.pallas_call(
    _prep_body,
    in_specs=[pl.BlockSpec((D, B), lambda: (0, 0))],
    out_specs=[
        pl.BlockSpec((B, D), lambda: (0, 0)),
        pl.BlockSpec((B, D), lambda: (0, 0)),
    ],
    out_shape=[
        jax.ShapeDtypeStruct((B, D), jnp.float32),
        jax.ShapeDtypeStruct((B, D), jnp.float32),
    ],
)


_TAIL_BLK = TAIL_Q * W // 128


def _tail_body(mem_ref, src_ref, idx_ref, p_ref, l_ref, out_ref, dot_ref):
  del mem_ref
  old = src_ref[...]
  colmask = lax.broadcasted_iota(jnp.int32, (1, 128), 1) < TAIL
  old = jnp.where(jnp.broadcast_to(colmask, (D, 128)), old, 0.0)
  cols = TAIL_Q * W + lax.broadcasted_iota(jnp.int32, (1, 128), 1)
  idx = idx_ref[...]
  m_hit = (idx == cols).astype(jnp.float32)
  p01 = p_ref[...]
  lg = l_ref[...]
  contract = (((0,), (0,)), ((), ()))
  p_sel = lax.dot_general(m_hit, p01, contract)
  lm = lax.dot_general(m_hit, lg, contract)
  anyhit = jnp.max(m_hit, axis=0, keepdims=True)
  blended = 0.9 * old + p_sel.T
  out_ref[...] = jnp.where(
      jnp.broadcast_to(anyhit > 0.0, (D, 128)), blended, old)
  term1 = jnp.sum(old * lm.T)
  anyhit_b = (idx >= TAIL_Q * W).astype(jnp.float32)
  s01 = jnp.sum(lg * p01, axis=1, keepdims=True)
  dot_ref[0, 0] = 0.9 * term1 + jnp.sum(anyhit_b * s01)


_tail_fix = pl.pallas_call(
    _tail_body,
    grid=(1,),
    in_specs=[
        pl.BlockSpec((D, 128), lambda i: (0, _TAIL_BLK)),
        pl.BlockSpec((D, 128), lambda i: (0, _TAIL_BLK)),
        pl.BlockSpec((B, 1), lambda i: (0, 0)),
        pl.BlockSpec((B, D), lambda i: (0, 0)),
        pl.BlockSpec((B, D), lambda i: (0, 0)),
    ],
    out_specs=[
        pl.BlockSpec((D, 128), lambda i: (0, _TAIL_BLK)),
        pl.BlockSpec(memory_space=pltpu.SMEM),
    ],
    out_shape=[
        jax.ShapeDtypeStruct((D, NUM_ROWS), jnp.float32),
        jax.ShapeDtypeStruct((1, 1), jnp.float32),
    ],
    input_output_aliases={0: 0},
)


_BLK = 2048
_NBLK = B // _BLK


def _loss_body(l_ref, dots_ref, dt_ref, loss_ref):
  i = pl.program_id(0)
  l = l_ref[...]
  part = jnp.sum(jnp.maximum(l, 0.0) + jnp.log1p(jnp.exp(-jnp.abs(l))))

  @pl.when(i == 0)
  def _():
    loss_ref[0, 0] = 0.0

  loss_ref[0, 0] += part

  @pl.when(i == _NBLK - 1)
  def _():
    loss_ref[0, 0] = (
        loss_ref[0, 0] - jnp.sum(dots_ref[...]) - dt_ref[0, 0]) / (B * D)


_loss = pl.pallas_call(
    _loss_body,
    grid=(_NBLK,),
    in_specs=[
        pl.BlockSpec((D, _BLK), lambda i: (0, i)),
        pl.BlockSpec((NC, NS, L16), lambda i: (0, 0, 0)),
        pl.BlockSpec(memory_space=pltpu.SMEM),
    ],
    out_specs=pl.BlockSpec(memory_space=pltpu.SMEM),
    out_shape=jax.ShapeDtypeStruct((1, 1), jnp.float32),
)


def kernel(logits, labels, index, targets_mem):
  del labels
  merge = _sc_merge()
  bank_t = targets_mem.T
  logits_t = logits.T
  p01_rows, l_rows = _prep(logits_t)
  new_bank, dots = merge(bank_t, index.reshape(B // 128, 128),
                         p01_rows.reshape(B * D // 128, 128),
                         l_rows.reshape(B * D // 128, 128))
  new_bank, dot_tail = _tail_fix(
      new_bank, bank_t, index.reshape(B, 1), p01_rows, l_rows)
  loss = _loss(logits_t, dots, dot_tail)
  return loss[0, 0], new_bank.T


# speedup vs baseline: 14.8843x; 1.0003x over previous
import functools

import jax
import jax.numpy as jnp
from jax import lax
from jax.experimental import pallas as pl
from jax.experimental.pallas import tpu as pltpu
from jax.experimental.pallas import tpu_sc as plsc

NUM_ROWS = 1_000_000
D = 32
B = 16_384
NC, NS = 2, 16
NW = NC * NS
W = 512
N_FULL = NUM_ROWS // W
TAIL = NUM_ROWS - N_FULL * W
TAIL_Q = N_FULL
TAIL_W = TAIL_Q % NW
_SHIFT = W.bit_length() - 1
L16 = 16
CAP_W = 4096
CAP_C = 2048


def _iota16():
  return lax.iota(jnp.int32, 16)


@functools.cache
def _sc_merge():
  mesh = plsc.VectorSubcoreMesh(
      core_axis_name="c", subcore_axis_name="s", num_cores=NC, num_subcores=NS)

  @functools.partial(
      pl.kernel,
      out_type=(
          jax.ShapeDtypeStruct((D, NUM_ROWS), jnp.float32),
          jax.ShapeDtypeStruct((NC, NS, L16), jnp.float32),
      ),
      mesh=mesh,
      compiler_params=pltpu.CompilerParams(needs_layout_passes=False),
      scratch_types=[
          pltpu.VMEM((B // 128, 128), jnp.int32),
          pltpu.VMEM((CAP_W + 16,), jnp.int32),
          pltpu.VMEM((CAP_W + 16,), jnp.int32),
          pltpu.VMEM((CAP_C + 128,), jnp.int32),
          pltpu.VMEM((CAP_C + 128,), jnp.int32),
          pltpu.VMEM((D, W + 8), jnp.float32),
          pltpu.VMEM((D, W + 8), jnp.float32),
          pltpu.VMEM((16, 256), jnp.float32),
          pltpu.VMEM((L16,), jnp.float32),
          pltpu.SemaphoreType.DMA,
          pltpu.SemaphoreType.DMA,
          pltpu.SemaphoreType.DMA,
          pltpu.SemaphoreType.DMA,
          pltpu.SemaphoreType.DMA,
          pltpu.VMEM_SHARED((NS, L16), jnp.float32),
      ],
  )
  def merge(bank, idx, pl_lines, out, dots, idx_all, r_hits, b_hits,
            ch_r, ch_b, buf_a, buf_b, stg, acc_v, stg_sem,
            in_sem_a, in_sem_b, out_sem_a, out_sem_b, shd):
    c = lax.axis_index("c")
    s = lax.axis_index("s")
    w = s * NC + c

    pltpu.sync_copy(idx, idx_all)

    zeros16i = jnp.zeros((16,), jnp.int32)
    sent16 = jnp.full((16,), jnp.int32(0x7FFFFFFF), jnp.int32)

    @pl.loop(0, (CAP_W + 16) // 16)
    def _(i):
      r_hits[pl.ds(i * 16, 16)] = sent16

    @pl.loop(0, (CAP_C + 128) // 16)
    def _(i):
      ch_r[pl.ds(i * 16, 16)] = zeros16i
      ch_b[pl.ds(i * 16, 16)] = zeros16i

    acc_v[...] = jnp.zeros((L16,), jnp.float32)

    def _compact(dst_a, dst_b, val_a, val_b, m, cnt, cap):
      mi = m.astype(jnp.int32)
      pos = cnt + jnp.cumsum(mi) - 1
      keep = m & (pos < cap)
      pos = jnp.where(keep, pos, jnp.full((16,), cap, jnp.int32))
      plsc.store_scatter(dst_a, [pos], val_a)
      plsc.store_scatter(dst_b, [pos], val_b)
      return jnp.minimum(cnt + jnp.sum(mi), cap)

    @pl.loop(0, B // 16, init_carry=jnp.int32(0))
    def n_w(i, cnt):
      row = lax.shift_right_logical(i, 3)
      col = (i % 8) * 16
      v = idx_all[row, pl.ds(col, 16)]
      m = (lax.shift_right_logical(v, _SHIFT) % NW) == w
      pos = _iota16() + i * 16
      return _compact(r_hits, b_hits, v, pos, m, cnt, CAP_W)

    plsc.subcore_barrier()

    ngrp = (n_w + 15) // 16

    def process_chunk(q, acc0, buf):
      @pl.loop(0, ngrp, init_carry=jnp.int32(0))
      def m(i, cnt):
        r16 = r_hits[pl.ds(i * 16, 16)]
        b16 = b_hits[pl.ds(i * 16, 16)]
        mm = lax.shift_right_logical(r16, _SHIFT) == q
        return _compact(ch_r, ch_b, r16 % W, b16, mm, cnt, CAP_C)

      @pl.loop(0, (m + 15) // 16, init_carry=acc0)
      def acc(p, a):
        base = p * 16
        rem = m - base
        r16 = ch_r[pl.ds(base, 16)]
        b16 = ch_b[pl.ds(base, 16)]
        rows = _iota16()
        msk = rows < rem
        gvec = lax.shift_right_logical(b16, 2)
        pltpu.async_copy(pl_lines.at[gvec], stg, stg_sem).wait()
        r16s = jnp.where(msk, r16, jnp.full((16,), W, jnp.int32))
        cbase = (b16 % 4) * D
        ag = a
        for cc in range(D):
          cvec = jnp.full((16,), cc, jnp.int32)
          old = plsc.load_gather(buf, [cvec, r16s])
          pv = plsc.load_gather(stg, [rows, cbase + cc])
          lv = plsc.load_gather(stg, [rows, cbase + (128 + cc)])
          t = jnp.where(msk, 0.9 * old + pv, old)
          plsc.store_scatter(buf, [cvec, r16s], t)
          ag = ag + jnp.where(msk, lv * t, 0.0)
        return ag

      return acc

    nq = jnp.where(w < (N_FULL % NW), N_FULL // NW + 1, N_FULL // NW)

    def _start_in(j, buf, sem):
      col0 = pl.multiple_of((w + NW * j) * W, W)
      pltpu.async_copy(bank.at[:, pl.ds(col0, W)], buf.at[:, pl.ds(0, W)],
                       sem)

    def _start_out(j, buf, sem):
      col0 = pl.multiple_of((w + NW * j) * W, W)
      pltpu.async_copy(buf.at[:, pl.ds(0, W)], out.at[:, pl.ds(col0, W)],
                       sem)

    def _drain(buf, sem):
      pltpu.make_async_copy(
          bank.at[:, pl.ds(0, W)], buf.at[:, pl.ds(0, W)], sem).wait()

    _start_in(jnp.int32(0), buf_a, in_sem_a)

    @pl.loop(0, nq)
    def _(j):
      even = (j % 2) == 0

      def _step(buf_c, in_c, out_c, buf_o, in_o, out_o):
        @pl.when(j + 1 < nq)
        def _():
          @pl.when(j >= 1)
          def _():
            _drain(buf_o, out_o)

          _start_in(j + 1, buf_o, in_o)

        _drain(buf_c, in_c)
        acc_v[...] = process_chunk(w + NW * j, acc_v[...], buf_c)
        _start_out(j, buf_c, out_c)

      @pl.when(even)
      def _():
        _step(buf_a, in_sem_a, out_sem_a, buf_b, in_sem_b, out_sem_b)

      @pl.when(jnp.logical_not(even))
      def _():
        _step(buf_b, in_sem_b, out_sem_b, buf_a, in_sem_a, out_sem_a)

    _drain(buf_a, out_sem_a)
    _drain(buf_b, out_sem_b)

    pltpu.sync_copy(acc_v, shd.at[s])
    plsc.subcore_barrier()

    @pl.when(s == 0)
    def _():
      pltpu.sync_copy(shd, dots.at[c])

  return merge


def _prep_body(lt_ref, p_ref, l_ref):
  lt = lt_ref[...]
  p_ref[...] = (0.1 * jax.nn.sigmoid(lt)).T
  l_ref[...] = lt.T


_prep = pl.pallas_call(
    _prep_body,
    in_specs=[pl.BlockSpec((D, B), lambda: (0, 0))],
    out_specs=[
        pl.BlockSpec((B, D), lambda: (0, 0)),
        pl.BlockSpec((B, D), lambda: (0, 0)),
    ],
    out_shape=[
        jax.ShapeDtypeStruct((B, D), jnp.float32),
        jax.ShapeDtypeStruct((B, D), jnp.float32),
    ],
)


_TAIL_BLK = TAIL_Q * W // 128


def _tail_body(mem_ref, src_ref, idx_ref, p_ref, l_ref, out_ref, dot_ref):
  del mem_ref
  old = src_ref[...]
  colmask = lax.broadcasted_iota(jnp.int32, (1, 128), 1) < TAIL
  old = jnp.where(jnp.broadcast_to(colmask, (D, 128)), old, 0.0)
  cols = TAIL_Q * W + lax.broadcasted_iota(jnp.int32, (1, 128), 1)
  idx = idx_ref[...]
  m_hit = (idx == cols).astype(jnp.float32)
  p01 = p_ref[...]
  lg = l_ref[...]
  contract = (((0,), (0,)), ((), ()))
  p_sel = lax.dot_general(m_hit, p01, contract)
  lm = lax.dot_general(m_hit, lg, contract)
  anyhit = jnp.max(m_hit, axis=0, keepdims=True)
  blended = 0.9 * old + p_sel.T
  out_ref[...] = jnp.where(
      jnp.broadcast_to(anyhit > 0.0, (D, 128)), blended, old)
  term1 = jnp.sum(old * lm.T)
  anyhit_b = (idx >= TAIL_Q * W).astype(jnp.float32)
  s01 = jnp.sum(lg * p01, axis=1, keepdims=True)
  dot_ref[0, 0] = 0.9 * term1 + jnp.sum(anyhit_b * s01)


_tail_fix = pl.pallas_call(
    _tail_body,
    grid=(1,),
    in_specs=[
        pl.BlockSpec((D, 128), lambda i: (0, _TAIL_BLK)),
        pl.BlockSpec((D, 128), lambda i: (0, _TAIL_BLK)),
        pl.BlockSpec((B, 1), lambda i: (0, 0)),
        pl.BlockSpec((B, D), lambda i: (0, 0)),
        pl.BlockSpec((B, D), lambda i: (0, 0)),
    ],
    out_specs=[
        pl.BlockSpec((D, 128), lambda i: (0, _TAIL_BLK)),
        pl.BlockSpec(memory_space=pltpu.SMEM),
    ],
    out_shape=[
        jax.ShapeDtypeStruct((D, NUM_ROWS), jnp.float32),
        jax.ShapeDtypeStruct((1, 1), jnp.float32),
    ],
    input_output_aliases={0: 0},
)


_BLK = 2048
_NBLK = B // _BLK


def _loss_body(l_ref, dots_ref, dt_ref, loss_ref):
  i = pl.program_id(0)
  l = l_ref[...]
  part = jnp.sum(jnp.maximum(l, 0.0) + jnp.log1p(jnp.exp(-jnp.abs(l))))

  @pl.when(i == 0)
  def _():
    loss_ref[0, 0] = 0.0

  loss_ref[0, 0] += part

  @pl.when(i == _NBLK - 1)
  def _():
    loss_ref[0, 0] = (
        loss_ref[0, 0] - jnp.sum(dots_ref[...]) - dt_ref[0, 0]) / (B * D)


_loss = pl.pallas_call(
    _loss_body,
    grid=(_NBLK,),
    in_specs=[
        pl.BlockSpec((D, _BLK), lambda i: (0, i)),
        pl.BlockSpec((NC, NS, L16), lambda i: (0, 0, 0)),
        pl.BlockSpec(memory_space=pltpu.SMEM),
    ],
    out_specs=pl.BlockSpec(memory_space=pltpu.SMEM),
    out_shape=jax.ShapeDtypeStruct((1, 1), jnp.float32),
)


def kernel(logits, labels, index, targets_mem):
  del labels
  merge = _sc_merge()
  bank_t = targets_mem.T
  logits_t = logits.T
  p01_rows, l_rows = _prep(logits_t)
  pl_lines = jnp.concatenate(
      [p01_rows.reshape(B * D // 128, 128),
       l_rows.reshape(B * D // 128, 128)], axis=1)
  new_bank, dots = merge(bank_t, index.reshape(B // 128, 128), pl_lines)
  new_bank, dot_tail = _tail_fix(
      new_bank, bank_t, index.reshape(B, 1), p01_rows, l_rows)
  loss = _loss(logits_t, dots, dot_tail)
  return loss[0, 0], new_bank.T
